# Initial kernel scaffold; baseline (speedup 1.0000x reference)
import jax
import jax.numpy as jnp
from jax.experimental import pallas as pl


def kernel(x, annotators, questions, annotator_embedding, question_embedding):
    raise NotImplementedError("write your pallas kernel here")


# speedup vs baseline: 1.3423x; 1.3423x over previous
import functools

import jax
import jax.numpy as jnp
from jax import lax
from jax.experimental import pallas as pl
from jax.experimental.pallas import tpu as pltpu
from jax.experimental.pallas import tpu_sc as plsc

D = 32
NC, NS = 2, 16
NW = NC * NS
CHUNK = 128


def _emb_body(n_chunks, ann_hbm, q_hbm, ai_hbm, qi_hbm, out_hbm,
              ai_v, qi_v, rows_a, rows_q, sem_a, sem_q):
    wid = lax.axis_index("s") * NC + lax.axis_index("c")
    base = wid * (n_chunks * CHUNK)

    def chunk(j, carry):
        cb = base + j * CHUNK
        pltpu.sync_copy(ai_hbm.at[pl.ds(cb, CHUNK)], ai_v)
        pltpu.sync_copy(qi_hbm.at[pl.ds(cb, CHUNK)], qi_v)
        ca = pltpu.async_copy(ann_hbm.at[ai_v], rows_a, sem_a)
        cq = pltpu.async_copy(q_hbm.at[qi_v], rows_q, sem_q)
        ca.wait()
        cq.wait()

        def row(i, c2):
            rows_a[i, pl.ds(0, 16)] = rows_a[i, pl.ds(0, 16)] + rows_q[i, pl.ds(0, 16)]
            rows_a[i, pl.ds(16, 16)] = rows_a[i, pl.ds(16, 16)] + rows_q[i, pl.ds(16, 16)]
            return c2

        lax.fori_loop(0, CHUNK, row, 0)
        pltpu.sync_copy(rows_a, out_hbm.at[pl.ds(cb, CHUNK)])
        return carry

    lax.fori_loop(0, n_chunks, chunk, 0)


def kernel(x, annotators, questions, annotator_embedding, question_embedding):
    B, S, XF = x.shape
    N = B * S
    assert N % (NW * CHUNK) == 0
    n_chunks = N // (NW * CHUNK)

    ai = annotators.reshape(N).astype(jnp.int32)
    qi = questions.reshape(N).astype(jnp.int32)

    mesh = plsc.VectorSubcoreMesh(core_axis_name="c", subcore_axis_name="s")
    emb = pl.kernel(
        functools.partial(_emb_body, n_chunks),
        out_type=jax.ShapeDtypeStruct((N, D), jnp.float32),
        mesh=mesh,
        compiler_params=pltpu.CompilerParams(use_tc_tiling_on_sc=False),
        scratch_types=[
            pltpu.VMEM((CHUNK,), jnp.int32),
            pltpu.VMEM((CHUNK,), jnp.int32),
            pltpu.VMEM((CHUNK, D), jnp.float32),
            pltpu.VMEM((CHUNK, D), jnp.float32),
            pltpu.SemaphoreType.DMA,
            pltpu.SemaphoreType.DMA,
        ],
    )(annotator_embedding, question_embedding, ai, qi)

    feature_x = jnp.concatenate([emb.reshape(B, S, D), x[:, :, 1:]], axis=-1)
    param_x = x[:, :, 1:]
    return feature_x, param_x
